# baseline (speedup 1.0000x reference)
import jax
import jax.numpy as jnp
from jax.experimental import pallas as pl
from jax.experimental.pallas import tpu as pltpu

N_ROWS = 200000
STATES = 128
S2 = 2 * STATES
SIT = 64
BLK = 25000
NBLK = N_ROWS // BLK
GBLK = 8


def _body(idx_ref, x_ref, wzt_ref, wrt_ref, wht_ref, uzt_ref, urt_ref,
          uht_ref, bz_ref, br_ref, bh_ref, gb_ref, gp_ref, st_ref,
          out_ref, dh_ref):
    i = pl.program_id(0)

    @pl.when(i == 0)
    def _compute_gru():
        rb = idx_ref[0] % GBLK
        rp = idx_ref[1] % GBLK
        h_b = gb_ref[pl.ds(rb, 1), :]
        h_p = gp_ref[pl.ds(rp, 1), :]
        h = jnp.concatenate([h_b, h_p], axis=1)
        xv = x_ref[...]
        dn = (((1,), (1,)), ((), ()))
        mv = lambda v, w: jax.lax.dot_general(
            v, w, dn, precision=jax.lax.Precision.HIGHEST)
        wx_z = mv(xv, wzt_ref[...])
        wx_r = mv(xv, wrt_ref[...])
        wx_h = mv(xv, wht_ref[...])
        z = jax.nn.sigmoid(wx_z + mv(h, uzt_ref[...]) + bz_ref[...])
        r = jax.nn.sigmoid(wx_r + mv(h, urt_ref[...]) - br_ref[...])
        m = jnp.tanh(wx_h + mv(r * h, uht_ref[...]) + bh_ref[...])
        hp_new = z * h + (1.0 - z) * m
        dh_ref[...] = hp_new - h

    out_ref[...] = st_ref[...]

    row_b = idx_ref[0]
    row_p = idx_ref[1]
    lo = i * BLK

    @pl.when(jnp.logical_and(row_b >= lo, row_b < lo + BLK))
    def _add_b():
        r = row_b - lo
        out_ref[pl.ds(r, 1), :] = out_ref[pl.ds(r, 1), :] + dh_ref[:, :STATES]

    @pl.when(jnp.logical_and(row_p >= lo, row_p < lo + BLK))
    def _add_p():
        r = row_p - lo
        out_ref[pl.ds(r, 1), :] = out_ref[pl.ds(r, 1), :] + dh_ref[:, STATES:]


def kernel(x, b, p, state, Wz, Wr, Wh, Uz, Ur, Uh, bz, br, bh):
    st = state.reshape(N_ROWS, STATES)
    idx = jnp.concatenate([b, p]).astype(jnp.int32)
    full = lambda arr: pl.BlockSpec(arr.shape, lambda i, s: (0,) * arr.ndim)
    grid_spec = pltpu.PrefetchScalarGridSpec(
        num_scalar_prefetch=1,
        grid=(NBLK,),
        in_specs=[
            full(jnp.zeros((1, SIT))),
            full(jnp.zeros((S2, SIT))),
            full(jnp.zeros((S2, SIT))),
            full(jnp.zeros((S2, SIT))),
            full(jnp.zeros((S2, S2))),
            full(jnp.zeros((S2, S2))),
            full(jnp.zeros((S2, S2))),
            full(jnp.zeros((1, S2))),
            full(jnp.zeros((1, S2))),
            full(jnp.zeros((1, S2))),
            pl.BlockSpec((GBLK, STATES), lambda i, s: (s[0] // GBLK, 0)),
            pl.BlockSpec((GBLK, STATES), lambda i, s: (s[1] // GBLK, 0)),
            pl.BlockSpec((BLK, STATES), lambda i, s: (i, 0)),
        ],
        out_specs=pl.BlockSpec((BLK, STATES), lambda i, s: (i, 0)),
        scratch_shapes=[pltpu.VMEM((1, S2), jnp.float32)],
    )
    out = pl.pallas_call(
        _body,
        grid_spec=grid_spec,
        out_shape=jax.ShapeDtypeStruct((N_ROWS, STATES), jnp.float32),
    )(idx, x.reshape(1, SIT), Wz, Wr, Wh, Uz, Ur, Uh,
      bz.reshape(1, S2), br.reshape(1, S2), bh.reshape(1, S2),
      st, st, st)
    return out.reshape(1, N_ROWS, STATES)


# speedup vs baseline: 1.0082x; 1.0082x over previous
import jax
import jax.numpy as jnp
from jax.experimental import pallas as pl
from jax.experimental.pallas import tpu as pltpu

N_ROWS = 200000
STATES = 128
S2 = 2 * STATES
SIT = 64
BLK = 25000
NBLK = N_ROWS // BLK
GBLK = 8


def _body(idx_ref, x_ref, wzt_ref, wrt_ref, wht_ref, uzt_ref, urt_ref,
          uht_ref, bz_ref, br_ref, bh_ref, gb_ref, gp_ref, st_ref,
          out_ref, dh_ref):
    i = pl.program_id(0)

    first_use = jnp.minimum(idx_ref[0], idx_ref[1]) // BLK

    @pl.when(i == first_use)
    def _compute_gru():
        rb = idx_ref[0] % GBLK
        rp = idx_ref[1] % GBLK
        h_b = gb_ref[pl.ds(rb, 1), :]
        h_p = gp_ref[pl.ds(rp, 1), :]
        h = jnp.concatenate([h_b, h_p], axis=1)
        xv = x_ref[...]
        dn = (((1,), (1,)), ((), ()))
        mv = lambda v, w: jax.lax.dot_general(
            v, w, dn, precision=jax.lax.Precision.HIGHEST)
        wx_z = mv(xv, wzt_ref[...])
        wx_r = mv(xv, wrt_ref[...])
        wx_h = mv(xv, wht_ref[...])
        z = jax.nn.sigmoid(wx_z + mv(h, uzt_ref[...]) + bz_ref[...])
        r = jax.nn.sigmoid(wx_r + mv(h, urt_ref[...]) - br_ref[...])
        m = jnp.tanh(wx_h + mv(r * h, uht_ref[...]) + bh_ref[...])
        hp_new = z * h + (1.0 - z) * m
        dh_ref[...] = hp_new - h

    out_ref[...] = st_ref[...]

    row_b = idx_ref[0]
    row_p = idx_ref[1]
    lo = i * BLK

    @pl.when(jnp.logical_and(row_b >= lo, row_b < lo + BLK))
    def _add_b():
        r = row_b - lo
        out_ref[pl.ds(r, 1), :] = out_ref[pl.ds(r, 1), :] + dh_ref[:, :STATES]

    @pl.when(jnp.logical_and(row_p >= lo, row_p < lo + BLK))
    def _add_p():
        r = row_p - lo
        out_ref[pl.ds(r, 1), :] = out_ref[pl.ds(r, 1), :] + dh_ref[:, STATES:]


def kernel(x, b, p, state, Wz, Wr, Wh, Uz, Ur, Uh, bz, br, bh):
    st = state.reshape(N_ROWS, STATES)
    idx = jnp.concatenate([b, p]).astype(jnp.int32)
    full = lambda arr: pl.BlockSpec(arr.shape, lambda i, s: (0,) * arr.ndim)
    grid_spec = pltpu.PrefetchScalarGridSpec(
        num_scalar_prefetch=1,
        grid=(NBLK,),
        in_specs=[
            full(jnp.zeros((1, SIT))),
            full(jnp.zeros((S2, SIT))),
            full(jnp.zeros((S2, SIT))),
            full(jnp.zeros((S2, SIT))),
            full(jnp.zeros((S2, S2))),
            full(jnp.zeros((S2, S2))),
            full(jnp.zeros((S2, S2))),
            full(jnp.zeros((1, S2))),
            full(jnp.zeros((1, S2))),
            full(jnp.zeros((1, S2))),
            pl.BlockSpec((GBLK, STATES), lambda i, s: (s[0] // GBLK, 0)),
            pl.BlockSpec((GBLK, STATES), lambda i, s: (s[1] // GBLK, 0)),
            pl.BlockSpec((BLK, STATES), lambda i, s: (i, 0)),
        ],
        out_specs=pl.BlockSpec((BLK, STATES), lambda i, s: (i, 0)),
        scratch_shapes=[pltpu.VMEM((1, S2), jnp.float32)],
    )
    out = pl.pallas_call(
        _body,
        grid_spec=grid_spec,
        out_shape=jax.ShapeDtypeStruct((N_ROWS, STATES), jnp.float32),
    )(idx, x.reshape(1, SIT), Wz, Wr, Wh, Uz, Ur, Uh,
      bz.reshape(1, S2), br.reshape(1, S2), bh.reshape(1, S2),
      st, st, st)
    return out.reshape(1, N_ROWS, STATES)
